# baseline (speedup 1.0000x reference)
import functools

import jax
import jax.numpy as jnp
from jax import lax
from jax.experimental import pallas as pl
from jax.experimental.pallas import tpu as pltpu
from jax.experimental.pallas import tpu_sc as plsc

D = 64
B = 16384
NC = 2
NS = 16
NW = NC * NS
BPW = B // NW
L = 16
NCH = BPW // L
TS = 8
TL = 128
NBUF = 4


def _dot_body(uembT, iembT, uidx, iidx, part_out, *refs):
    idx_u, idx_i = refs[0], refs[1]
    su = refs[2:2 + NBUF]
    si = refs[2 + NBUF:2 + 2 * NBUF]
    accv = refs[2 + 2 * NBUF]
    sems = refs[3 + 2 * NBUF:]

    c = lax.axis_index("c")
    s = lax.axis_index("s")
    wid = s * NC + c
    base = wid * BPW

    pltpu.sync_copy(uidx.at[pl.ds(base, BPW)], idx_u)
    pltpu.sync_copy(iidx.at[pl.ds(base, BPW)], idx_i)

    iota = lax.broadcasted_iota(jnp.int32, (L,), 0)

    def enqueue(p_u, p_i, j):
        cu = pl.multiple_of((p_u // TL) * TL, TL)
        ci = pl.multiple_of((p_i // TL) * TL, TL)
        pltpu.make_async_copy(
            uembT.at[:, pl.ds(cu, TL)], su[j], sems[j]).start()
        pltpu.make_async_copy(
            iembT.at[:, pl.ds(ci, TL)], si[j], sems[j]).start()

    def wait_slot(j):
        for _ in range(2):
            pltpu.make_async_copy(
                uembT.at[:, pl.ds(0, TL)], su[j], sems[j]).wait()

    def consume(pvu, pvi, j, jj, accs):
        accs = list(accs)
        wait_slot(jj)
        q_u = jnp.full((L,), pvu[j] % TL, jnp.int32)
        q_i = jnp.full((L,), pvi[j] % TL, jnp.int32)
        for g in range(4):
            d_vec = g * L + iota
            uvals = plsc.load_gather(su[jj], [d_vec, q_u])
            ivals = plsc.load_gather(si[jj], [d_vec, q_i])
            accs[g] = accs[g] + uvals * ivals
        return tuple(accs)

    vu0 = idx_u[pl.ds(0, L)]
    vi0 = idx_i[pl.ds(0, L)]
    for j in range(NBUF):
        enqueue(vu0[j], vi0[j], j)

    zero = jnp.zeros((L,), jnp.float32)

    def chunk_body(g, carry):
        a0, a1, a2, a3, pvu, pvi = carry
        vu = idx_u[pl.ds(g * L, L)]
        vi = idx_i[pl.ds(g * L, L)]
        accs = (a0, a1, a2, a3)
        for j in range(L):
            lane = (j + NBUF) % L
            src_u = pvu if j < NBUF else vu
            src_i = pvi if j < NBUF else vi
            accs = consume(src_u, src_i, lane, j % NBUF, accs)
            enqueue(vu[j], vi[j], j % NBUF)
        return accs + (vu, vi)

    a0, a1, a2, a3, pvu, pvi = lax.fori_loop(
        0, NCH, chunk_body, (zero, zero, zero, zero, vu0, vi0))

    accs = (a0, a1, a2, a3)
    for j in range(NBUF):
        accs = consume(pvu, pvi, j + NBUF, j, accs)

    accv[...] = (accs[0] + accs[1]) + (accs[2] + accs[3])
    pltpu.sync_copy(accv, part_out.at[pl.ds(wid * L, L)])


_dot_call = functools.partial(
    pl.kernel,
    out_type=jax.ShapeDtypeStruct((NW * L,), jnp.float32),
    mesh=plsc.VectorSubcoreMesh(core_axis_name="c", subcore_axis_name="s"),
    compiler_params=pltpu.CompilerParams(
        use_tc_tiling_on_sc=True, needs_layout_passes=False),
    scratch_types=(
        [pltpu.VMEM((BPW,), jnp.int32)] * 2
        + [pltpu.VMEM((D, TL), jnp.float32)] * (2 * NBUF)
        + [pltpu.VMEM((L,), jnp.float32)]
        + [pltpu.SemaphoreType.DMA] * NBUF
    ),
)(_dot_body)


def _bias_body(ubt, ibt, uidx, iidx,
               bsum_out,
               idx_u, idx_i, bu, bi,
               sem_bu, sem_bi):
    c = lax.axis_index("c")
    s = lax.axis_index("s")
    wid = s * NC + c
    base = wid * BPW

    pltpu.sync_copy(uidx.at[pl.ds(base, BPW)], idx_u)
    pltpu.sync_copy(iidx.at[pl.ds(base, BPW)], idx_i)
    cp_bu = pltpu.async_copy(ubt.at[idx_u], bu, sem_bu)
    cp_bi = pltpu.async_copy(ibt.at[idx_i], bi, sem_bi)
    cp_bu.wait()
    cp_bi.wait()
    for j in range(BPW // L):
        sl = pl.ds(j * L, L)
        bu[sl] = bu[sl] + bi[sl]
    pltpu.sync_copy(bu, bsum_out.at[pl.ds(base, BPW)])


_bias_call = functools.partial(
    pl.kernel,
    out_type=jax.ShapeDtypeStruct((B,), jnp.float32),
    mesh=plsc.VectorSubcoreMesh(core_axis_name="c", subcore_axis_name="s"),
    compiler_params=pltpu.CompilerParams(use_tc_tiling_on_sc=False),
    scratch_types=[
        pltpu.VMEM((BPW,), jnp.int32),
        pltpu.VMEM((BPW,), jnp.int32),
        pltpu.VMEM((BPW,), jnp.float32),
        pltpu.VMEM((BPW,), jnp.float32),
        pltpu.SemaphoreType.DMA,
        pltpu.SemaphoreType.DMA,
    ],
)(_bias_body)


def _finish_body(part_ref, bsum_ref, out_ref):
    scalar = jnp.sum(part_ref[...])
    out_ref[...] = jnp.maximum(bsum_ref[...] + scalar, 0.0)


def kernel(user_emb, user_bias_tbl, item_emb, item_bias_tbl, inputs):
    idx = inputs.astype(jnp.int32).T
    uidx, iidx = idx[0], idx[1]
    bb = lax.broadcasted_iota(jnp.int32, (B,), 0)
    sk_u, _ = lax.sort_key_val(uidx, bb)
    sk_i, _ = lax.sort_key_val(iidx, bb)
    uidx = jnp.where(sk_u < 0, sk_u, uidx)
    iidx = jnp.where(sk_i < 0, sk_i, iidx)
    partials = _dot_call(user_emb.T, item_emb.T, uidx, iidx)
    bsum = _bias_call(
        user_bias_tbl.T.reshape(-1), item_bias_tbl.T.reshape(-1),
        uidx, iidx)
    out = pl.pallas_call(
        _finish_body,
        out_shape=jax.ShapeDtypeStruct((B // 128, 128), jnp.float32),
    )(partials.reshape(NW, L), bsum.reshape(B // 128, 128))
    return out.reshape(B, 1)


# speedup vs baseline: 1.8604x; 1.8604x over previous
import functools

import jax
import jax.numpy as jnp
from jax import lax
from jax.experimental import pallas as pl
from jax.experimental.pallas import tpu as pltpu
from jax.experimental.pallas import tpu_sc as plsc

D = 64
B = 16384
NC = 2
NS = 16
NW = NC * NS
BPW = B // NW
L = 16
NCH = BPW // L
TL = 128
NB = 10
K = 8

TILE_BYTES = D * TL * 4


def _gather_body(tabT, pos_s, fidx_s, newf_s, out, *refs):
    (idx_p, idx_f, idx_n, stk, colv, sem, sem_out) = refs

    c = lax.axis_index("c")
    s = lax.axis_index("s")
    wid = s * NC + c
    base = wid * BPW

    pltpu.sync_copy(pos_s.at[pl.ds(base, BPW)], idx_p)
    pltpu.sync_copy(fidx_s.at[pl.ds(base, BPW)], idx_f)
    pltpu.sync_copy(newf_s.at[pl.ds(base, BPW)], idx_n)

    iota = lax.broadcasted_iota(jnp.int32, (L,), 0)

    def enqueue(pos_j, nf_j, fx_j):
        col = pl.multiple_of((pos_j // TL) * TL, TL)
        slot = (fx_j - 1) % NB

        @pl.when(nf_j == 1)
        def _():
            pltpu.make_async_copy(
                tabT.at[:, pl.ds(col, TL)], stk.at[slot], sem).start()

    def consume(e_scal, pos_j, fx_j, oslot, waited):
        def wbody(_, w):
            pltpu.make_async_copy(
                tabT.at[:, pl.ds(0, TL)], stk.at[0], sem).wait()
            return w
        lax.fori_loop(0, fx_j - waited, wbody, 0)
        waited = jnp.maximum(waited, fx_j)
        q = jnp.full((L,), pos_j % TL, jnp.int32)
        slotv = jnp.full((L,), (fx_j - 1) % NB, jnp.int32)
        for g in range(4):
            vals = plsc.load_gather(stk, [slotv, g * L + iota, q])
            colv[oslot, pl.ds(g * L, L)] = vals
        return waited

    def flush(e_next, half):
        @pl.when(e_next >= 0)
        def _():
            dst = pl.multiple_of(base + e_next - 7, 8)
            pltpu.make_async_copy(
                colv.at[pl.ds(half * 8, 8), :],
                out.at[pl.ds(dst, 8), :], sem_out).start()

    def chunk_body(g, carry):
        waited, ppos, pfx = carry
        pos_v = idx_p[pl.ds(g * L, L)]
        fx_v = idx_f[pl.ds(g * L, L)]
        nf_v = idx_n[pl.ds(g * L, L)]
        for j in range(L):
            e = g * L + j - K
            lane = (j + K) % L
            spos = ppos if j < K else pos_v
            sfx = pfx if j < K else fx_v
            if j < K:
                waited = lax.cond(
                    e >= 0,
                    lambda w: _consume_wrap(e, spos[lane], sfx[lane],
                                            lane, w),
                    lambda w: w,
                    waited)
            else:
                waited = _consume_wrap(e, spos[lane], sfx[lane], lane,
                                       waited)
            if lane == 7:
                flush(e, 0)
            if lane == 15:
                flush(e, 1)
            enqueue(pos_v[j], nf_v[j], fx_v[j])
        return (waited, pos_v, fx_v)

    def _consume_wrap(e, pos_j, fx_j, oslot, waited):
        if oslot in (0, K):
            @pl.when(e >= 16)
            def _():
                pltpu.make_async_copy(
                    colv.at[pl.ds(0, 8), :], out.at[pl.ds(0, 8), :],
                    sem_out).wait()
        return consume(e, pos_j, fx_j, oslot, waited)

    zero = jnp.int32(0)
    pos0 = idx_p[pl.ds(0, L)]
    waited, ppos, pfx = lax.fori_loop(
        0, NCH, chunk_body, (zero, pos0, pos0))

    for j in range(K):
        e = BPW - K + j
        waited = _consume_wrap(e, ppos[j + K], pfx[j + K], j + K, waited)
    flush(jnp.int32(BPW - 1), 1)
    for _ in range(2):
        pltpu.make_async_copy(
            colv.at[pl.ds(0, 8), :], out.at[pl.ds(0, 8), :], sem_out).wait()


_gather_call = functools.partial(
    pl.kernel,
    out_type=jax.ShapeDtypeStruct((B, TL), jnp.float32),
    mesh=plsc.VectorSubcoreMesh(core_axis_name="c", subcore_axis_name="s"),
    compiler_params=pltpu.CompilerParams(
        use_tc_tiling_on_sc=True, needs_layout_passes=False),
    scratch_types=[
        pltpu.VMEM((BPW,), jnp.int32),
        pltpu.VMEM((BPW,), jnp.int32),
        pltpu.VMEM((BPW,), jnp.int32),
        pltpu.VMEM((NB, D, TL), jnp.float32),
        pltpu.VMEM((L, TL), jnp.float32),
        pltpu.SemaphoreType.DMA,
        pltpu.SemaphoreType.DMA,
    ],
)(_gather_body)


def _combine_body(ucols, vcols, rank_u, rank_v, ubt, ibt, uidx, iidx,
                  part_out, bsum_out,
                  ru0, ru1, rv0, rv1, rows_u, rows_i,
                  idx_u, idx_i, bu, bi, accv,
                  sem_u, sem_i, sem_b):
    c = lax.axis_index("c")
    s = lax.axis_index("s")
    wid = s * NC + c
    base = wid * BPW
    H = BPW // 2

    pltpu.sync_copy(rank_u.at[pl.ds(base, H)], ru0)
    pltpu.sync_copy(rank_u.at[pl.ds(base + H, H)], ru1)
    pltpu.sync_copy(rank_v.at[pl.ds(base, H)], rv0)
    pltpu.sync_copy(rank_v.at[pl.ds(base + H, H)], rv1)
    pltpu.sync_copy(uidx.at[pl.ds(base, BPW)], idx_u)
    pltpu.sync_copy(iidx.at[pl.ds(base, BPW)], idx_i)
    cp_bu = pltpu.async_copy(ubt.at[idx_u], bu, sem_b)
    cp_bi = pltpu.async_copy(ibt.at[idx_i], bi, sem_b)

    zero = jnp.zeros((L,), jnp.float32)
    accs = [zero, zero, zero, zero]
    for h, (ru, rv) in enumerate(((ru0, rv0), (ru1, rv1))):
        cu = pltpu.async_copy(ucols.at[ru], rows_u, sem_u)
        cv = pltpu.async_copy(vcols.at[rv], rows_i, sem_i)
        cu.wait()
        cv.wait()

        def body(r, acc):
            a0, a1, a2, a3 = acc
            a0 = a0 + rows_u[r, pl.ds(0, L)] * rows_i[r, pl.ds(0, L)]
            a1 = a1 + rows_u[r, pl.ds(L, L)] * rows_i[r, pl.ds(L, L)]
            a2 = a2 + rows_u[r, pl.ds(2 * L, L)] * rows_i[r, pl.ds(2 * L, L)]
            a3 = a3 + rows_u[r, pl.ds(3 * L, L)] * rows_i[r, pl.ds(3 * L, L)]
            return (a0, a1, a2, a3)

        accs = list(lax.fori_loop(0, H, body, tuple(accs)))

    accv[...] = (accs[0] + accs[1]) + (accs[2] + accs[3])
    pltpu.sync_copy(accv, part_out.at[pl.ds(wid * L, L)])

    cp_bu.wait()
    cp_bi.wait()
    for j in range(BPW // L):
        sl = pl.ds(j * L, L)
        bu[sl] = bu[sl] + bi[sl]
    pltpu.sync_copy(bu, bsum_out.at[pl.ds(base, BPW)])


_combine_call = functools.partial(
    pl.kernel,
    out_type=(
        jax.ShapeDtypeStruct((NW * L,), jnp.float32),
        jax.ShapeDtypeStruct((B,), jnp.float32),
    ),
    mesh=plsc.VectorSubcoreMesh(core_axis_name="c", subcore_axis_name="s"),
    compiler_params=pltpu.CompilerParams(use_tc_tiling_on_sc=False),
    scratch_types=(
        [pltpu.VMEM((BPW // 2,), jnp.int32)] * 4
        + [pltpu.VMEM((BPW // 2, TL), jnp.float32)] * 2
        + [pltpu.VMEM((BPW,), jnp.int32)] * 2
        + [pltpu.VMEM((BPW,), jnp.float32)] * 2
        + [pltpu.VMEM((L,), jnp.float32)]
        + [pltpu.SemaphoreType.DMA] * 3
    ),
)(_combine_body)


def _finish_body(part_ref, bsum_ref, out_ref):
    scalar = jnp.sum(part_ref[...])
    out_ref[...] = jnp.maximum(bsum_ref[...] + scalar, 0.0)


def _prep(pos, bb):
    srt, perm = lax.sort_key_val(pos, bb)
    col = srt // TL
    prev = jnp.concatenate([col[:1] - 1, col[:-1]])
    newf = ((col != prev) | (bb % BPW == 0)).astype(jnp.int32)
    fidx = jnp.cumsum(newf.reshape(NW, BPW), axis=1).reshape(-1)
    fidx = fidx.astype(jnp.int32)
    _, rank = lax.sort_key_val(perm, bb)
    return srt, fidx, newf, rank


def kernel(user_emb, user_bias_tbl, item_emb, item_bias_tbl, inputs):
    idx = inputs.astype(jnp.int32).T
    uidx, iidx = idx[0], idx[1]
    bb = lax.broadcasted_iota(jnp.int32, (B,), 0)
    spos_u, fidx_u, newf_u, rank_u = _prep(uidx, bb)
    spos_v, fidx_v, newf_v, rank_v = _prep(iidx, bb)
    ucols = _gather_call(user_emb.T, spos_u, fidx_u, newf_u)
    vcols = _gather_call(item_emb.T, spos_v, fidx_v, newf_v)
    partials, bsum = _combine_call(
        ucols, vcols, rank_u, rank_v,
        user_bias_tbl.T.reshape(-1), item_bias_tbl.T.reshape(-1),
        uidx, iidx)
    out = pl.pallas_call(
        _finish_body,
        out_shape=jax.ShapeDtypeStruct((B // 128, 128), jnp.float32),
    )(partials.reshape(NW, L), bsum.reshape(B // 128, 128))
    return out.reshape(B, 1)


# speedup vs baseline: 2.0142x; 1.0827x over previous
import functools

import jax
import jax.numpy as jnp
from jax import lax
from jax.experimental import pallas as pl
from jax.experimental.pallas import tpu as pltpu
from jax.experimental.pallas import tpu_sc as plsc

D = 64
B = 16384
NC = 2
NS = 16
NW = NC * NS
BPW = B // NW
L = 16
NCH = BPW // L
TL = 128
NB = 14
K = 12

TILE_BYTES = D * TL * 4


def _gather_body(tabT, pos_s, fidx_s, newf_s, out, *refs):
    (idx_p, idx_f, idx_n, stk, colv, sem, sem_out) = refs

    c = lax.axis_index("c")
    s = lax.axis_index("s")
    wid = s * NC + c
    base = wid * BPW

    pltpu.sync_copy(pos_s.at[pl.ds(base, BPW)], idx_p)
    pltpu.sync_copy(fidx_s.at[pl.ds(base, BPW)], idx_f)
    pltpu.sync_copy(newf_s.at[pl.ds(base, BPW)], idx_n)

    iota = lax.broadcasted_iota(jnp.int32, (L,), 0)

    def enqueue(pos_j, nf_j, fx_j):
        col = pl.multiple_of((pos_j // TL) * TL, TL)
        slot = (fx_j - 1) % NB

        @pl.when(nf_j == 1)
        def _():
            pltpu.make_async_copy(
                tabT.at[:, pl.ds(col, TL)], stk.at[slot], sem).start()

    def consume(e_scal, pos_j, fx_j, oslot, waited):
        def wbody(_, w):
            pltpu.make_async_copy(
                tabT.at[:, pl.ds(0, TL)], stk.at[0], sem).wait()
            return w
        lax.fori_loop(0, fx_j - waited, wbody, 0)
        waited = jnp.maximum(waited, fx_j)
        q = jnp.full((L,), pos_j % TL, jnp.int32)
        slotv = jnp.full((L,), (fx_j - 1) % NB, jnp.int32)
        for g in range(4):
            vals = plsc.load_gather(stk, [slotv, g * L + iota, q])
            colv[oslot, pl.ds(g * L, L)] = vals
        return waited

    def flush(e_next, half):
        @pl.when(e_next >= 0)
        def _():
            dst = pl.multiple_of(base + e_next - 7, 8)
            pltpu.make_async_copy(
                colv.at[pl.ds(half * 8, 8), :],
                out.at[pl.ds(dst, 8), :], sem_out).start()

    def chunk_body(g, carry):
        waited, ppos, pfx = carry
        pos_v = idx_p[pl.ds(g * L, L)]
        fx_v = idx_f[pl.ds(g * L, L)]
        nf_v = idx_n[pl.ds(g * L, L)]
        for j in range(L):
            e = g * L + j - K
            lane = (j + L - K) % L
            spos = ppos if j < K else pos_v
            sfx = pfx if j < K else fx_v
            if j < K:
                waited = lax.cond(
                    e >= 0,
                    lambda w: _consume_wrap(e, spos[lane], sfx[lane],
                                            lane, w),
                    lambda w: w,
                    waited)
            else:
                waited = _consume_wrap(e, spos[lane], sfx[lane], lane,
                                       waited)
            if lane == 7:
                flush(e, 0)
            if lane == 15:
                flush(e, 1)
            enqueue(pos_v[j], nf_v[j], fx_v[j])
        return (waited, pos_v, fx_v)

    def _consume_wrap(e, pos_j, fx_j, oslot, waited):
        if oslot in (0, 8):
            @pl.when(e >= 16)
            def _():
                pltpu.make_async_copy(
                    colv.at[pl.ds(0, 8), :], out.at[pl.ds(0, 8), :],
                    sem_out).wait()
        return consume(e, pos_j, fx_j, oslot, waited)

    zero = jnp.int32(0)
    pos0 = idx_p[pl.ds(0, L)]
    waited, ppos, pfx = lax.fori_loop(
        0, NCH, chunk_body, (zero, pos0, pos0))

    for j in range(K):
        e = BPW - K + j
        lane = L - K + j
        waited = _consume_wrap(e, ppos[lane], pfx[lane], lane, waited)
        if lane == 7:
            flush(jnp.int32(e), 0)
        if lane == 15:
            flush(jnp.int32(e), 1)
    for _ in range(2):
        pltpu.make_async_copy(
            colv.at[pl.ds(0, 8), :], out.at[pl.ds(0, 8), :], sem_out).wait()


_gather_call = functools.partial(
    pl.kernel,
    out_type=jax.ShapeDtypeStruct((B, TL), jnp.float32),
    mesh=plsc.VectorSubcoreMesh(core_axis_name="c", subcore_axis_name="s"),
    compiler_params=pltpu.CompilerParams(
        use_tc_tiling_on_sc=True, needs_layout_passes=False),
    scratch_types=[
        pltpu.VMEM((BPW,), jnp.int32),
        pltpu.VMEM((BPW,), jnp.int32),
        pltpu.VMEM((BPW,), jnp.int32),
        pltpu.VMEM((NB, D, TL), jnp.float32),
        pltpu.VMEM((L, TL), jnp.float32),
        pltpu.SemaphoreType.DMA,
        pltpu.SemaphoreType.DMA,
    ],
)(_gather_body)


def _combine_body(ucols, vcols, rank_u, rank_v, ubt, ibt, uidx, iidx,
                  part_out, bsum_out,
                  ru0, ru1, rv0, rv1, rows_u, rows_i,
                  idx_u, idx_i, bu, bi, accv,
                  sem_u, sem_i, sem_b):
    c = lax.axis_index("c")
    s = lax.axis_index("s")
    wid = s * NC + c
    base = wid * BPW
    H = BPW // 2

    pltpu.sync_copy(rank_u.at[pl.ds(base, H)], ru0)
    pltpu.sync_copy(rank_u.at[pl.ds(base + H, H)], ru1)
    pltpu.sync_copy(rank_v.at[pl.ds(base, H)], rv0)
    pltpu.sync_copy(rank_v.at[pl.ds(base + H, H)], rv1)
    pltpu.sync_copy(uidx.at[pl.ds(base, BPW)], idx_u)
    pltpu.sync_copy(iidx.at[pl.ds(base, BPW)], idx_i)
    cp_bu = pltpu.async_copy(ubt.at[idx_u], bu, sem_b)
    cp_bi = pltpu.async_copy(ibt.at[idx_i], bi, sem_b)

    zero = jnp.zeros((L,), jnp.float32)
    accs = [zero, zero, zero, zero]
    for h, (ru, rv) in enumerate(((ru0, rv0), (ru1, rv1))):
        cu = pltpu.async_copy(ucols.at[ru], rows_u, sem_u)
        cv = pltpu.async_copy(vcols.at[rv], rows_i, sem_i)
        cu.wait()
        cv.wait()

        def body(r, acc):
            a0, a1, a2, a3 = acc
            a0 = a0 + rows_u[r, pl.ds(0, L)] * rows_i[r, pl.ds(0, L)]
            a1 = a1 + rows_u[r, pl.ds(L, L)] * rows_i[r, pl.ds(L, L)]
            a2 = a2 + rows_u[r, pl.ds(2 * L, L)] * rows_i[r, pl.ds(2 * L, L)]
            a3 = a3 + rows_u[r, pl.ds(3 * L, L)] * rows_i[r, pl.ds(3 * L, L)]
            return (a0, a1, a2, a3)

        accs = list(lax.fori_loop(0, H, body, tuple(accs)))

    accv[...] = (accs[0] + accs[1]) + (accs[2] + accs[3])
    pltpu.sync_copy(accv, part_out.at[pl.ds(wid * L, L)])

    cp_bu.wait()
    cp_bi.wait()
    for j in range(BPW // L):
        sl = pl.ds(j * L, L)
        bu[sl] = bu[sl] + bi[sl]
    pltpu.sync_copy(bu, bsum_out.at[pl.ds(base, BPW)])


_combine_call = functools.partial(
    pl.kernel,
    out_type=(
        jax.ShapeDtypeStruct((NW * L,), jnp.float32),
        jax.ShapeDtypeStruct((B,), jnp.float32),
    ),
    mesh=plsc.VectorSubcoreMesh(core_axis_name="c", subcore_axis_name="s"),
    compiler_params=pltpu.CompilerParams(use_tc_tiling_on_sc=False),
    scratch_types=(
        [pltpu.VMEM((BPW // 2,), jnp.int32)] * 4
        + [pltpu.VMEM((BPW // 2, TL), jnp.float32)] * 2
        + [pltpu.VMEM((BPW,), jnp.int32)] * 2
        + [pltpu.VMEM((BPW,), jnp.float32)] * 2
        + [pltpu.VMEM((L,), jnp.float32)]
        + [pltpu.SemaphoreType.DMA] * 3
    ),
)(_combine_body)


def _finish_body(part_ref, bsum_ref, out_ref):
    scalar = jnp.sum(part_ref[...])
    out_ref[...] = jnp.maximum(bsum_ref[...] + scalar, 0.0)


def _prep(pos, bb):
    srt, perm = lax.sort_key_val(pos, bb)
    col = srt // TL
    prev = jnp.concatenate([col[:1] - 1, col[:-1]])
    newf = ((col != prev) | (bb % BPW == 0)).astype(jnp.int32)
    fidx = jnp.cumsum(newf.reshape(NW, BPW), axis=1).reshape(-1)
    fidx = fidx.astype(jnp.int32)
    _, rank = lax.sort_key_val(perm, bb)
    return srt, fidx, newf, rank


def kernel(user_emb, user_bias_tbl, item_emb, item_bias_tbl, inputs):
    idx = inputs.astype(jnp.int32).T
    uidx, iidx = idx[0], idx[1]
    bb = lax.broadcasted_iota(jnp.int32, (B,), 0)
    spos_u, fidx_u, newf_u, rank_u = _prep(uidx, bb)
    spos_v, fidx_v, newf_v, rank_v = _prep(iidx, bb)
    ucols = _gather_call(user_emb.T, spos_u, fidx_u, newf_u)
    vcols = _gather_call(item_emb.T, spos_v, fidx_v, newf_v)
    partials, bsum = _combine_call(
        ucols, vcols, rank_u, rank_v,
        user_bias_tbl.T.reshape(-1), item_bias_tbl.T.reshape(-1),
        uidx, iidx)
    out = pl.pallas_call(
        _finish_body,
        out_shape=jax.ShapeDtypeStruct((B // 128, 128), jnp.float32),
    )(partials.reshape(NW, L), bsum.reshape(B // 128, 128))
    return out.reshape(B, 1)
